# baseline (speedup 1.0000x reference)
import functools

import jax
import jax.numpy as jnp
from jax import lax
from jax.experimental import pallas as pl
from jax.experimental.pallas import tpu as pltpu
from jax.experimental.pallas import tpu_sc as plsc

H = 32
W = 32
D = 256
B = 8
L = 16
NC = 2


def _sc_body(col_hbm, row_hbm, out_hbm, rowv, chunk, sem):
    h = lax.axis_index("s") * NC + lax.axis_index("c")
    pltpu.sync_copy(row_hbm.at[h], rowv)
    pltpu.sync_copy(col_hbm.at[pl.ds(0, W)], chunk.at[:, pl.ds(0, D)])
    for j in range(D // L):
        v = rowv[pl.ds(j * L, L)]
        for w in range(W):
            chunk[w, pl.ds(D + j * L, L)] = v
    copies = [
        pltpu.async_copy(chunk, out_hbm.at[b, h], sem) for b in range(B)
    ]
    for c in copies:
        c.wait()


def _sc_call(row_embed, col_embed):
    mesh = plsc.VectorSubcoreMesh(core_axis_name="c", subcore_axis_name="s")
    f = pl.kernel(
        _sc_body,
        out_type=jax.ShapeDtypeStruct((B, H, W, 2 * D), jnp.float32),
        mesh=mesh,
        scratch_types=[
            pltpu.VMEM((D,), jnp.float32),
            pltpu.VMEM((W, 2 * D), jnp.float32),
            pltpu.SemaphoreType.DMA,
        ],
    )
    return f(col_embed, row_embed)


def kernel(x, row_embed, col_embed):
    out = _sc_call(row_embed, col_embed)
    return jnp.transpose(out, (0, 3, 1, 2))


# speedup vs baseline: 4.4575x; 4.4575x over previous
import jax
import jax.numpy as jnp
from jax.experimental import pallas as pl

H = 32
W = 32
D = 256


BB = 2


def _body(col_ref, row_ref, out_ref):
    col = col_ref[...]
    for bb in range(BB):
        for h in range(H):
            out_ref[bb, h, :, :D] = col
            out_ref[bb, h, :, D:] = jnp.broadcast_to(
                row_ref[h, :][None, :], (W, D)
            )


def kernel(x, row_embed, col_embed):
    b = x.shape[0]
    out = pl.pallas_call(
        _body,
        grid=(b // BB,),
        in_specs=[
            pl.BlockSpec((W, D), lambda i: (0, 0)),
            pl.BlockSpec((H, D), lambda i: (0, 0)),
        ],
        out_specs=pl.BlockSpec((BB, H, W, 2 * D), lambda i: (i, 0, 0, 0)),
        out_shape=jax.ShapeDtypeStruct((b, H, W, 2 * D), jnp.float32),
    )(col_embed, row_embed)
    return jnp.transpose(out, (0, 3, 1, 2))
